# baseline (speedup 1.0000x reference)
import functools

import jax
import jax.numpy as jnp
from jax import lax
from jax.experimental import pallas as pl
from jax.experimental.pallas import tpu as pltpu
from jax.experimental.pallas import tpu_sc as plsc

_PAR = pltpu.CompilerParams(dimension_semantics=("parallel",))
_KP = 24
_NW = 32

_K = 20
_BLK = 256


def _leaky(v):
    return jnp.where(v >= 0, v, 0.2 * v)


def _dot(a, b, precision=jax.lax.Precision.HIGHEST):
    return jax.lax.dot_general(a, b, (((1,), (0,)), ((), ())),
                               precision=precision,
                               preferred_element_type=jnp.float32)


def _knn_body(ptb_ref, ptT_ref, idx_ref, dist_ref, ds_ref):
    blk = ptb_ref.shape[0]
    n = ptT_ref.shape[1]
    ptb = ptb_ref[:, :]
    ptT = ptT_ref[:, :]
    sq_col = jnp.sum(ptb * ptb, axis=1, keepdims=True)
    sq_row = jnp.sum(ptT * ptT, axis=0, keepdims=True)
    inner = _dot(ptb, ptT, precision=jax.lax.Precision.DEFAULT)
    d2 = sq_col + sq_row - 2.0 * inner
    iota = jax.lax.broadcasted_iota(jnp.int32, (blk, n), 1)
    cur = d2
    for k in range(_K):
        m = jnp.min(cur, axis=1, keepdims=True)
        j = jnp.min(jnp.where(cur == m, iota, n), axis=1, keepdims=True)
        oh = iota == j
        dsq = jnp.zeros((blk, 1), jnp.float32)
        for c in range(3):
            pj_c = jnp.sum(jnp.where(oh, ptT[c:c + 1, :], 0.0), axis=1,
                           keepdims=True)
            rel_c = pj_c - ptb[:, c:c + 1]
            dsq = dsq + rel_c * rel_c
        dk = jnp.sqrt(jnp.maximum(dsq, 1e-12))
        idx_ref[:, k:k + 1] = j
        dist_ref[:, k:k + 1] = dk
        cur = jnp.where(oh, jnp.float32(jnp.inf), cur)
    ds_ref[:, :] = jnp.sum(dist_ref[:, :], axis=1, keepdims=True)


def _knn(pt):
    n = pt.shape[0]
    blk = min(_BLK, n)
    return pl.pallas_call(
        _knn_body,
        grid=(n // blk,),
        in_specs=[pl.BlockSpec((blk, 3), lambda i: (i, 0)),
                  pl.BlockSpec((3, n), lambda i: (0, 0))],
        out_specs=(pl.BlockSpec((blk, _K), lambda i: (i, 0)),
                   pl.BlockSpec((blk, _K), lambda i: (i, 0)),
                   pl.BlockSpec((blk, 1), lambda i: (i, 0))),
        out_shape=(jax.ShapeDtypeStruct((n, _K), jnp.int32),
                   jax.ShapeDtypeStruct((n, _K), jnp.float32),
                   jax.ShapeDtypeStruct((n, 1), jnp.float32)),
        compiler_params=_PAR,
    )(pt, pt.T)


def _lin_body(xp_ref, wg_ref, wb_ref, b_ref, g_ref, base_ref):
    xp = xp_ref[:, :]
    g_ref[:, :] = _dot(xp, wg_ref[:, :])
    base_ref[:, :] = _dot(xp, wb_ref[:, :]) + b_ref[:, :]


def _tree_max(vals):
    while len(vals) > 1:
        nxt = [jnp.maximum(vals[i], vals[i + 1])
               for i in range(0, len(vals) - 1, 2)]
        if len(vals) % 2:
            nxt.append(vals[-1])
        vals = nxt
    return vals[0]


@functools.lru_cache(maxsize=None)
def _gmax_sc(bn, cout):
    pts = bn // _NW
    ppf = 2 if cout > 256 else 4
    ngr = pts // ppf
    mesh = plsc.VectorSubcoreMesh(core_axis_name="c", subcore_axis_name="s")
    nchunk = cout // 16

    def body(g_hbm, base_hbm, idx_hbm, dist_hbm, wd_hbm, out_hbm,
             idx_v, dist_v, wd_v, base_v, out_v, rows_a, rows_b,
             sem_a, sem_b):
        wid = lax.axis_index("s") * 2 + lax.axis_index("c")
        p0 = wid * pts
        pltpu.sync_copy(idx_hbm.at[pl.ds(p0 * _K, pts * _K)],
                        idx_v.at[pl.ds(0, pts * _K)])
        pltpu.sync_copy(idx_hbm.at[pl.ds(p0 * _K, ppf * _K)],
                        idx_v.at[pl.ds(pts * _K, ppf * _K)])
        pltpu.sync_copy(dist_hbm.at[pl.ds(p0 * _K, pts * _K)], dist_v)
        pltpu.sync_copy(wd_hbm, wd_v)
        pltpu.sync_copy(base_hbm.at[pl.ds(p0, pts)], base_v)

        def fire(q, buf, sem):
            pltpu.async_copy(
                g_hbm.at[idx_v.at[pl.ds(q * ppf * _K, ppf * _K)]], buf, sem)

        def drain(buf, sem):
            pltpu.make_async_copy(
                g_hbm.at[idx_v.at[pl.ds(0, ppf * _K)]], buf, sem).wait()

        def compute(q, rows):
            for pp in range(ppf):
                p = q * ppf + pp
                dks = [plsc.load_gather(
                    dist_v, [jnp.full((16,), p * _K + k, jnp.int32)])
                    for k in range(_K)]
                for c in range(nchunk):
                    sl = pl.ds(c * 16, 16)
                    wdc = wd_v[sl]
                    acc = _tree_max(
                        [rows[pp * _K + k, sl] + dks[k] * wdc
                         for k in range(_K)])
                    o = base_v[p, sl] + acc
                    out_v[p, sl] = jnp.where(o >= 0, o, 0.2 * o)

        fire(0, rows_a, sem_a)

        def pair(i, carry):
            qa, qb = 2 * i, 2 * i + 1
            fire(qb, rows_b, sem_b)
            drain(rows_a, sem_a)
            compute(qa, rows_a)
            fire(qb + 1, rows_a, sem_a)
            drain(rows_b, sem_b)
            compute(qb, rows_b)
            return carry

        lax.fori_loop(0, ngr // 2, pair, 0)
        drain(rows_a, sem_a)
        pltpu.sync_copy(out_v, out_hbm.at[pl.ds(p0, pts)])

    return pl.kernel(
        body,
        out_type=jax.ShapeDtypeStruct((bn, cout), jnp.float32),
        mesh=mesh,
        compiler_params=pltpu.CompilerParams(needs_layout_passes=False,
                                             use_tc_tiling_on_sc=False),
        scratch_types=[
            pltpu.VMEM(((pts + ppf) * _K,), jnp.int32),
            pltpu.VMEM((pts * _K,), jnp.float32),
            pltpu.VMEM((cout,), jnp.float32),
            pltpu.VMEM((pts, cout), jnp.float32),
            pltpu.VMEM((pts, cout), jnp.float32),
            pltpu.VMEM((ppf * _K, cout), jnp.float32),
            pltpu.VMEM((ppf * _K, cout), jnp.float32),
            pltpu.SemaphoreType.DMA,
            pltpu.SemaphoreType.DMA,
        ],
    )


def _conv(xp, idx, dist, w, b, c):
    bsz, n = xp.shape[0], xp.shape[1]
    cout = w.shape[0]
    wxi = w[:, :c]
    wxj = w[:, c:2 * c]
    wrel = w[:, 2 * c:2 * c + 3]
    wd = w[:, 2 * c + 3]
    wg = jnp.concatenate([wxj, wrel], axis=1).T
    wb = jnp.concatenate([wxi - wxj, -wrel], axis=1).T
    g, base = jax.vmap(
        lambda xpb: pl.pallas_call(
            _lin_body,
            out_shape=(jax.ShapeDtypeStruct((n, cout), jnp.float32),
                       jax.ShapeDtypeStruct((n, cout), jnp.float32)),
        )(xpb, wg, wb, b[None, :]))(xp)
    boff = (jnp.arange(bsz, dtype=jnp.int32) * n)[:, None, None]
    out = _gmax_sc(bsz * n, cout)(
        g.reshape(bsz * n, cout),
        base.reshape(bsz * n, cout),
        (idx + boff).reshape(bsz * n * _K),
        dist.reshape(bsz * n * _K),
        wd,
    )
    return out.reshape(bsz, n, cout)


def _idis_body(dsc_ref, dsr_ref, ptT_ref, feat_ref, pos_ref, out_ref):
    n = dsc_ref.shape[0]
    n_keep = out_ref.shape[0]
    v_col = dsc_ref[:, :]
    v_row = dsr_ref[:, :]
    iota_sub = jax.lax.broadcasted_iota(jnp.int32, (n, n), 0)
    iota_lane = jax.lax.broadcasted_iota(jnp.int32, (n, n), 1)
    beats = (v_col > v_row) | ((v_col == v_row) & (iota_sub < iota_lane))
    rank = jnp.sum(beats.astype(jnp.int32), axis=0, keepdims=True)
    sel_iota = jax.lax.broadcasted_iota(jnp.int32, (n_keep, n), 0)
    sel = rank == sel_iota
    for c in range(3):
        pos_ref[:, c:c + 1] = jnp.sum(
            jnp.where(sel, ptT_ref[c:c + 1, :], 0.0), axis=1, keepdims=True)
    out_ref[:, :] = _dot(sel.astype(jnp.float32), feat_ref[:, :])


def _idis(ds, pt, feat, n_keep):
    n = pt.shape[0]
    c = feat.shape[1]
    return pl.pallas_call(
        _idis_body,
        out_shape=(jax.ShapeDtypeStruct((n_keep, 3), jnp.float32),
                   jax.ShapeDtypeStruct((n_keep, c), jnp.float32)),
    )(ds, ds.T, pt.T, feat)


def _interp_body(pq_ref, psT_ref, fs_ref, out_ref):
    nq = pq_ref.shape[0]
    m_src = psT_ref.shape[1]
    pq = pq_ref[:, :]
    psT = psT_ref[:, :]
    sqq = jnp.sum(pq * pq, axis=1, keepdims=True)
    sqs = jnp.sum(psT * psT, axis=0, keepdims=True)
    inner = _dot(pq, psT.astype(jnp.float32),
                 precision=jax.lax.Precision.DEFAULT)
    d2 = sqq + sqs - 2.0 * inner
    iota = jax.lax.broadcasted_iota(jnp.int32, (nq, m_src), 1)
    cur = d2
    acc = jnp.zeros((nq, fs_ref.shape[1]), jnp.float32)
    wsum = jnp.zeros((nq, 1), jnp.float32)
    for _ in range(3):
        mv = jnp.min(cur, axis=1, keepdims=True)
        j = jnp.min(jnp.where(cur == mv, iota, m_src), axis=1, keepdims=True)
        oh = (iota == j).astype(jnp.float32)
        nb = _dot(oh, fs_ref[:, :])
        w = 1.0 / jnp.maximum(mv, 1e-16)
        acc = acc + nb * w
        wsum = wsum + w
        cur = jnp.where(iota == j, jnp.float32(jnp.inf), cur)
    out_ref[:, :] = acc / wsum


def _interp(fs, ps, pq):
    nq = pq.shape[0]
    c = fs.shape[1]
    return pl.pallas_call(
        _interp_body,
        out_shape=jax.ShapeDtypeStruct((nq, c), jnp.float32),
    )(pq, ps.T, fs)


def _head_body(x_ref, w4_ref, b4_ref, w5_ref, b5_ref, out_ref):
    h = _leaky(_dot(x_ref[:, :], w4_ref[:, :]) + b4_ref[:, :])
    out_ref[:, :] = _dot(h, w5_ref[:, :]) + b5_ref[:, :]


def _head(x, w4, b4, w5, b5):
    n = x.shape[0]
    return pl.pallas_call(
        _head_body,
        out_shape=jax.ShapeDtypeStruct((n, w5.shape[0]), jnp.float32),
    )(x, w4.T, b4[None, :], w5.T, b5[None, :])


def kernel(x, W0, b0, W1, b1, W2, b2, W3, b3, W4, b4, W5, b5, W6, b6,
           Wc4, bc4, Wc5, bc5):
    pt = jnp.transpose(x, (0, 2, 1))
    cat = lambda *a: jnp.concatenate(a, axis=2)

    idx_a, dist_a, ds_a = jax.vmap(_knn)(pt)
    x0 = _conv(cat(pt, pt), idx_a, dist_a, W0, b0, 3)
    x1 = _conv(cat(x0, pt), idx_a, dist_a, W1, b1, 64)

    pt2, x2in = jax.vmap(lambda d, p, f: _idis(d, p, f, 512))(ds_a, pt, x1)
    idx_b, dist_b, ds_b = jax.vmap(_knn)(pt2)
    x2 = _conv(cat(x2in, pt2), idx_b, dist_b, W2, b2, 128)

    pt3, x3in = jax.vmap(lambda d, p, f: _idis(d, p, f, 256))(ds_b, pt2, x2)
    idx_c, dist_c, _ = jax.vmap(_knn)(pt3)
    x3 = _conv(cat(x3in, pt3), idx_c, dist_c, W3, b3, 256)

    i43 = jax.vmap(_interp)(x3, pt3, pt2)
    x4 = _conv(cat(i43, x2, pt2), idx_b, dist_b, W4, b4, 768)

    i54 = jax.vmap(_interp)(x4, pt2, pt)
    x5 = _conv(cat(i54, x1, pt), idx_a, dist_a, W5, b5, 384)
    x6 = _conv(cat(x5, pt), idx_a, dist_a, W6, b6, 256)

    out = jax.vmap(lambda xb: _head(xb, Wc4, bc4, Wc5, bc5))(x6)
    return jnp.transpose(out, (0, 2, 1))


# speedup vs baseline: 1.0753x; 1.0753x over previous
import functools

import jax
import jax.numpy as jnp
from jax import lax
from jax.experimental import pallas as pl
from jax.experimental.pallas import tpu as pltpu
from jax.experimental.pallas import tpu_sc as plsc

_PAR = pltpu.CompilerParams(dimension_semantics=("parallel",))
_KP = 24
_NW = 32

_K = 20
_BLK = 256


def _leaky(v):
    return jnp.where(v >= 0, v, 0.2 * v)


def _dot(a, b, precision=jax.lax.Precision.HIGHEST):
    return jax.lax.dot_general(a, b, (((1,), (0,)), ((), ())),
                               precision=precision,
                               preferred_element_type=jnp.float32)


def _knn_body(ptb_ref, ptT_ref, idx_ref, dist_ref, ds_ref):
    blk = ptb_ref.shape[0]
    n = ptT_ref.shape[1]
    ptb = ptb_ref[:, :]
    ptT = ptT_ref[:, :]
    sq_col = jnp.sum(ptb * ptb, axis=1, keepdims=True)
    sq_row = jnp.sum(ptT * ptT, axis=0, keepdims=True)
    inner = _dot(ptb, ptT, precision=jax.lax.Precision.DEFAULT)
    d2 = sq_col + sq_row - 2.0 * inner
    iota = jax.lax.broadcasted_iota(jnp.int32, (blk, n), 1)
    cur = d2
    for k in range(_K):
        m = jnp.min(cur, axis=1, keepdims=True)
        j = jnp.min(jnp.where(cur == m, iota, n), axis=1, keepdims=True)
        oh = iota == j
        dsq = jnp.zeros((blk, 1), jnp.float32)
        for c in range(3):
            pj_c = jnp.sum(jnp.where(oh, ptT[c:c + 1, :], 0.0), axis=1,
                           keepdims=True)
            rel_c = pj_c - ptb[:, c:c + 1]
            dsq = dsq + rel_c * rel_c
        dk = jnp.sqrt(jnp.maximum(dsq, 1e-12))
        idx_ref[:, k:k + 1] = j
        dist_ref[:, k:k + 1] = dk
        cur = jnp.where(oh, jnp.float32(jnp.inf), cur)
    ds_ref[:, :] = jnp.sum(dist_ref[:, :], axis=1, keepdims=True)


def _knn(pt):
    n = pt.shape[0]
    blk = min(_BLK, n)
    return pl.pallas_call(
        _knn_body,
        grid=(n // blk,),
        in_specs=[pl.BlockSpec((blk, 3), lambda i: (i, 0)),
                  pl.BlockSpec((3, n), lambda i: (0, 0))],
        out_specs=(pl.BlockSpec((blk, _K), lambda i: (i, 0)),
                   pl.BlockSpec((blk, _K), lambda i: (i, 0)),
                   pl.BlockSpec((blk, 1), lambda i: (i, 0))),
        out_shape=(jax.ShapeDtypeStruct((n, _K), jnp.int32),
                   jax.ShapeDtypeStruct((n, _K), jnp.float32),
                   jax.ShapeDtypeStruct((n, 1), jnp.float32)),
        compiler_params=_PAR,
    )(pt, pt.T)


def _lin_body(xp_ref, wg_ref, wb_ref, b_ref, g_ref, base_ref):
    xp = xp_ref[:, :]
    g_ref[:, :] = _dot(xp, wg_ref[:, :])
    base_ref[:, :] = _dot(xp, wb_ref[:, :]) + b_ref[:, :]


def _blocked_max(vals):
    acc = None
    for i in range(0, len(vals), 4):
        blk = vals[i:i + 4]
        while len(blk) > 1:
            blk = [jnp.maximum(blk[j], blk[j + 1])
                   for j in range(0, len(blk) - 1, 2)] + (
                [blk[-1]] if len(blk) % 2 else [])
        acc = blk[0] if acc is None else jnp.maximum(acc, blk[0])
    return acc


@functools.lru_cache(maxsize=None)
def _gmax_sc(bn, cout):
    pts = bn // _NW
    ppf = 2
    ngr = pts // ppf
    mesh = plsc.VectorSubcoreMesh(core_axis_name="c", subcore_axis_name="s")
    nchunk = cout // 16

    def body(g_hbm, base_hbm, idx_hbm, dist_hbm, wd_hbm, out_hbm,
             idx_v, dist_v, wd_v, base_v, out_v, rows_a, rows_b,
             sem_a, sem_b):
        wid = lax.axis_index("s") * 2 + lax.axis_index("c")
        p0 = wid * pts
        pltpu.sync_copy(idx_hbm.at[pl.ds(p0 * _K, pts * _K)],
                        idx_v.at[pl.ds(0, pts * _K)])
        pltpu.sync_copy(idx_hbm.at[pl.ds(p0 * _K, ppf * _K)],
                        idx_v.at[pl.ds(pts * _K, ppf * _K)])
        pltpu.sync_copy(dist_hbm.at[pl.ds(p0 * _K, pts * _K)], dist_v)
        pltpu.sync_copy(wd_hbm, wd_v)
        pltpu.sync_copy(base_hbm.at[pl.ds(p0, pts)], base_v)

        def fire(q, buf, sem):
            pltpu.async_copy(
                g_hbm.at[idx_v.at[pl.ds(q * ppf * _K, ppf * _K)]], buf, sem)

        def drain(buf, sem):
            pltpu.make_async_copy(
                g_hbm.at[idx_v.at[pl.ds(0, ppf * _K)]], buf, sem).wait()

        def compute(q, rows):
            for pp in range(ppf):
                p = q * ppf + pp
                dks = [plsc.load_gather(
                    dist_v, [jnp.full((16,), p * _K + k, jnp.int32)])
                    for k in range(_K)]
                for c in range(nchunk):
                    sl = pl.ds(c * 16, 16)
                    wdc = wd_v[sl]
                    acc = _blocked_max(
                        [rows[pp * _K + k, sl] + dks[k] * wdc
                         for k in range(_K)])
                    o = base_v[p, sl] + acc
                    out_v[p, sl] = jnp.where(o >= 0, o, 0.2 * o)

        fire(0, rows_a, sem_a)

        def pair(i, carry):
            qa, qb = 2 * i, 2 * i + 1
            fire(qb, rows_b, sem_b)
            drain(rows_a, sem_a)
            compute(qa, rows_a)
            fire(qb + 1, rows_a, sem_a)
            drain(rows_b, sem_b)
            compute(qb, rows_b)
            return carry

        lax.fori_loop(0, ngr // 2, pair, 0)
        drain(rows_a, sem_a)
        pltpu.sync_copy(out_v, out_hbm.at[pl.ds(p0, pts)])

    return pl.kernel(
        body,
        out_type=jax.ShapeDtypeStruct((bn, cout), jnp.float32),
        mesh=mesh,
        compiler_params=pltpu.CompilerParams(needs_layout_passes=False,
                                             use_tc_tiling_on_sc=False),
        scratch_types=[
            pltpu.VMEM(((pts + ppf) * _K,), jnp.int32),
            pltpu.VMEM((pts * _K,), jnp.float32),
            pltpu.VMEM((cout,), jnp.float32),
            pltpu.VMEM((pts, cout), jnp.float32),
            pltpu.VMEM((pts, cout), jnp.float32),
            pltpu.VMEM((ppf * _K, cout), jnp.float32),
            pltpu.VMEM((ppf * _K, cout), jnp.float32),
            pltpu.SemaphoreType.DMA,
            pltpu.SemaphoreType.DMA,
        ],
    )


def _conv(xp, idx, dist, w, b, c):
    bsz, n = xp.shape[0], xp.shape[1]
    cout = w.shape[0]
    wxi = w[:, :c]
    wxj = w[:, c:2 * c]
    wrel = w[:, 2 * c:2 * c + 3]
    wd = w[:, 2 * c + 3]
    wg = jnp.concatenate([wxj, wrel], axis=1).T
    wb = jnp.concatenate([wxi - wxj, -wrel], axis=1).T
    g, base = jax.vmap(
        lambda xpb: pl.pallas_call(
            _lin_body,
            out_shape=(jax.ShapeDtypeStruct((n, cout), jnp.float32),
                       jax.ShapeDtypeStruct((n, cout), jnp.float32)),
        )(xpb, wg, wb, b[None, :]))(xp)
    boff = (jnp.arange(bsz, dtype=jnp.int32) * n)[:, None, None]
    out = _gmax_sc(bsz * n, cout)(
        g.reshape(bsz * n, cout),
        base.reshape(bsz * n, cout),
        (idx + boff).reshape(bsz * n * _K),
        dist.reshape(bsz * n * _K),
        wd,
    )
    return out.reshape(bsz, n, cout)


def _idis_body(dsc_ref, dsr_ref, ptT_ref, feat_ref, pos_ref, out_ref):
    n = dsc_ref.shape[0]
    n_keep = out_ref.shape[0]
    v_col = dsc_ref[:, :]
    v_row = dsr_ref[:, :]
    iota_sub = jax.lax.broadcasted_iota(jnp.int32, (n, n), 0)
    iota_lane = jax.lax.broadcasted_iota(jnp.int32, (n, n), 1)
    beats = (v_col > v_row) | ((v_col == v_row) & (iota_sub < iota_lane))
    rank = jnp.sum(beats.astype(jnp.int32), axis=0, keepdims=True)
    sel_iota = jax.lax.broadcasted_iota(jnp.int32, (n_keep, n), 0)
    sel = rank == sel_iota
    for c in range(3):
        pos_ref[:, c:c + 1] = jnp.sum(
            jnp.where(sel, ptT_ref[c:c + 1, :], 0.0), axis=1, keepdims=True)
    out_ref[:, :] = _dot(sel.astype(jnp.float32), feat_ref[:, :])


def _idis(ds, pt, feat, n_keep):
    n = pt.shape[0]
    c = feat.shape[1]
    return pl.pallas_call(
        _idis_body,
        out_shape=(jax.ShapeDtypeStruct((n_keep, 3), jnp.float32),
                   jax.ShapeDtypeStruct((n_keep, c), jnp.float32)),
    )(ds, ds.T, pt.T, feat)


def _interp_body(pq_ref, psT_ref, fs_ref, out_ref):
    nq = pq_ref.shape[0]
    m_src = psT_ref.shape[1]
    pq = pq_ref[:, :]
    psT = psT_ref[:, :]
    sqq = jnp.sum(pq * pq, axis=1, keepdims=True)
    sqs = jnp.sum(psT * psT, axis=0, keepdims=True)
    inner = _dot(pq, psT.astype(jnp.float32),
                 precision=jax.lax.Precision.DEFAULT)
    d2 = sqq + sqs - 2.0 * inner
    iota = jax.lax.broadcasted_iota(jnp.int32, (nq, m_src), 1)
    cur = d2
    acc = jnp.zeros((nq, fs_ref.shape[1]), jnp.float32)
    wsum = jnp.zeros((nq, 1), jnp.float32)
    for _ in range(3):
        mv = jnp.min(cur, axis=1, keepdims=True)
        j = jnp.min(jnp.where(cur == mv, iota, m_src), axis=1, keepdims=True)
        oh = (iota == j).astype(jnp.float32)
        nb = _dot(oh, fs_ref[:, :])
        w = 1.0 / jnp.maximum(mv, 1e-16)
        acc = acc + nb * w
        wsum = wsum + w
        cur = jnp.where(iota == j, jnp.float32(jnp.inf), cur)
    out_ref[:, :] = acc / wsum


def _interp(fs, ps, pq):
    nq = pq.shape[0]
    c = fs.shape[1]
    return pl.pallas_call(
        _interp_body,
        out_shape=jax.ShapeDtypeStruct((nq, c), jnp.float32),
    )(pq, ps.T, fs)


def _head_body(x_ref, w4_ref, b4_ref, w5_ref, b5_ref, out_ref):
    h = _leaky(_dot(x_ref[:, :], w4_ref[:, :]) + b4_ref[:, :])
    out_ref[:, :] = _dot(h, w5_ref[:, :]) + b5_ref[:, :]


def _head(x, w4, b4, w5, b5):
    n = x.shape[0]
    return pl.pallas_call(
        _head_body,
        out_shape=jax.ShapeDtypeStruct((n, w5.shape[0]), jnp.float32),
    )(x, w4.T, b4[None, :], w5.T, b5[None, :])


def kernel(x, W0, b0, W1, b1, W2, b2, W3, b3, W4, b4, W5, b5, W6, b6,
           Wc4, bc4, Wc5, bc5):
    pt = jnp.transpose(x, (0, 2, 1))
    cat = lambda *a: jnp.concatenate(a, axis=2)

    idx_a, dist_a, ds_a = jax.vmap(_knn)(pt)
    x0 = _conv(cat(pt, pt), idx_a, dist_a, W0, b0, 3)
    x1 = _conv(cat(x0, pt), idx_a, dist_a, W1, b1, 64)

    pt2, x2in = jax.vmap(lambda d, p, f: _idis(d, p, f, 512))(ds_a, pt, x1)
    idx_b, dist_b, ds_b = jax.vmap(_knn)(pt2)
    x2 = _conv(cat(x2in, pt2), idx_b, dist_b, W2, b2, 128)

    pt3, x3in = jax.vmap(lambda d, p, f: _idis(d, p, f, 256))(ds_b, pt2, x2)
    idx_c, dist_c, _ = jax.vmap(_knn)(pt3)
    x3 = _conv(cat(x3in, pt3), idx_c, dist_c, W3, b3, 256)

    i43 = jax.vmap(_interp)(x3, pt3, pt2)
    x4 = _conv(cat(i43, x2, pt2), idx_b, dist_b, W4, b4, 768)

    i54 = jax.vmap(_interp)(x4, pt2, pt)
    x5 = _conv(cat(i54, x1, pt), idx_a, dist_a, W5, b5, 384)
    x6 = _conv(cat(x5, pt), idx_a, dist_a, W6, b6, 256)

    out = jax.vmap(lambda xb: _head(xb, Wc4, bc4, Wc5, bc5))(x6)
    return jnp.transpose(out, (0, 2, 1))


# speedup vs baseline: 1.1051x; 1.0277x over previous
import functools

import jax
import jax.numpy as jnp
from jax import lax
from jax.experimental import pallas as pl
from jax.experimental.pallas import tpu as pltpu
from jax.experimental.pallas import tpu_sc as plsc

_PAR = pltpu.CompilerParams(dimension_semantics=("parallel",))
_KP = 24
_NW = 32

_K = 20
_BLK = 256


def _leaky(v):
    return jnp.where(v >= 0, v, 0.2 * v)


def _dot(a, b, precision=jax.lax.Precision.HIGHEST):
    return jax.lax.dot_general(a, b, (((1,), (0,)), ((), ())),
                               precision=precision,
                               preferred_element_type=jnp.float32)


def _knn_body(ptb_ref, ptT_ref, idx_ref, dist_ref, ds_ref):
    blk = ptb_ref.shape[0]
    n = ptT_ref.shape[1]
    ptb = ptb_ref[:, :]
    ptT = ptT_ref[:, :]
    sq_col = jnp.sum(ptb * ptb, axis=1, keepdims=True)
    sq_row = jnp.sum(ptT * ptT, axis=0, keepdims=True)
    inner = _dot(ptb, ptT, precision=jax.lax.Precision.DEFAULT)
    d2 = sq_col + sq_row - 2.0 * inner
    iota = jax.lax.broadcasted_iota(jnp.int32, (blk, n), 1)
    cur = d2
    for k in range(_K):
        m = jnp.min(cur, axis=1, keepdims=True)
        j = jnp.min(jnp.where(cur == m, iota, n), axis=1, keepdims=True)
        oh = iota == j
        dsq = jnp.zeros((blk, 1), jnp.float32)
        for c in range(3):
            pj_c = jnp.sum(jnp.where(oh, ptT[c:c + 1, :], 0.0), axis=1,
                           keepdims=True)
            rel_c = pj_c - ptb[:, c:c + 1]
            dsq = dsq + rel_c * rel_c
        dk = jnp.sqrt(jnp.maximum(dsq, 1e-12))
        idx_ref[:, k:k + 1] = j
        dist_ref[:, k:k + 1] = dk
        cur = jnp.where(oh, jnp.float32(jnp.inf), cur)
    ds_ref[:, :] = jnp.sum(dist_ref[:, :], axis=1, keepdims=True)


def _knn(pt):
    n = pt.shape[0]
    blk = min(_BLK, n)
    return pl.pallas_call(
        _knn_body,
        grid=(n // blk,),
        in_specs=[pl.BlockSpec((blk, 3), lambda i: (i, 0)),
                  pl.BlockSpec((3, n), lambda i: (0, 0))],
        out_specs=(pl.BlockSpec((blk, _K), lambda i: (i, 0)),
                   pl.BlockSpec((blk, _K), lambda i: (i, 0)),
                   pl.BlockSpec((blk, 1), lambda i: (i, 0))),
        out_shape=(jax.ShapeDtypeStruct((n, _K), jnp.int32),
                   jax.ShapeDtypeStruct((n, _K), jnp.float32),
                   jax.ShapeDtypeStruct((n, 1), jnp.float32)),
        compiler_params=_PAR,
    )(pt, pt.T)


def _lin_body(xp_ref, wg_ref, wb_ref, b_ref, g_ref, base_ref):
    xp = xp_ref[:, :]
    g_ref[:, :] = _dot(xp, wg_ref[:, :])
    base_ref[:, :] = _dot(xp, wb_ref[:, :]) + b_ref[:, :]


def _blocked_max(vals):
    acc = None
    for i in range(0, len(vals), 4):
        blk = vals[i:i + 4]
        while len(blk) > 1:
            blk = [jnp.maximum(blk[j], blk[j + 1])
                   for j in range(0, len(blk) - 1, 2)] + (
                [blk[-1]] if len(blk) % 2 else [])
        acc = blk[0] if acc is None else jnp.maximum(acc, blk[0])
    return acc


@functools.lru_cache(maxsize=None)
def _gmax_sc(bn, cout):
    pts = bn // _NW
    mesh = plsc.VectorSubcoreMesh(core_axis_name="c", subcore_axis_name="s")
    nchunk = cout // 16

    def body(g_hbm, base_hbm, idx_hbm, dist_hbm, wd_hbm, out_hbm,
             idx_v, dist_v, wd_v, base_v, out_v, rows_a, rows_b,
             sem_a, sem_b):
        wid = lax.axis_index("s") * 2 + lax.axis_index("c")
        p0 = wid * pts
        pltpu.sync_copy(idx_hbm.at[pl.ds(p0, pts)], idx_v.at[pl.ds(0, pts)])
        pltpu.sync_copy(idx_hbm.at[pl.ds(p0, 1)], idx_v.at[pl.ds(pts, 1)])
        pltpu.sync_copy(dist_hbm.at[pl.ds(p0 * _KP, pts * _KP)], dist_v)
        pltpu.sync_copy(wd_hbm, wd_v)
        pltpu.sync_copy(base_hbm.at[pl.ds(p0, pts)], base_v)

        def fire(p, buf, sem):
            pltpu.async_copy(g_hbm.at[idx_v.at[p]], buf, sem)

        def drain(buf, sem):
            pltpu.make_async_copy(g_hbm.at[idx_v.at[0]], buf, sem).wait()

        def compute(p, rows):
            dks = [plsc.load_gather(
                dist_v, [jnp.full((16,), p * _KP + k, jnp.int32)])
                for k in range(_KP)]
            for c in range(nchunk):
                sl = pl.ds(c * 16, 16)
                wdc = wd_v[sl]
                acc = _blocked_max(
                    [rows[k, sl] + dks[k] * wdc for k in range(_KP)])
                o = base_v[p, sl] + acc
                out_v[p, sl] = jnp.where(o >= 0, o, 0.2 * o)

        fire(0, rows_a, sem_a)

        def pair(i, carry):
            pa, pb = 2 * i, 2 * i + 1
            fire(pb, rows_b, sem_b)
            drain(rows_a, sem_a)
            compute(pa, rows_a)
            fire(pb + 1, rows_a, sem_a)
            drain(rows_b, sem_b)
            compute(pb, rows_b)
            return carry

        lax.fori_loop(0, pts // 2, pair, 0)
        drain(rows_a, sem_a)
        pltpu.sync_copy(out_v, out_hbm.at[pl.ds(p0, pts)])

    return pl.kernel(
        body,
        out_type=jax.ShapeDtypeStruct((bn, cout), jnp.float32),
        mesh=mesh,
        compiler_params=pltpu.CompilerParams(needs_layout_passes=False,
                                             use_tc_tiling_on_sc=False),
        scratch_types=[
            pltpu.VMEM((pts + 1, _KP), jnp.int32),
            pltpu.VMEM((pts * _KP,), jnp.float32),
            pltpu.VMEM((cout,), jnp.float32),
            pltpu.VMEM((pts, cout), jnp.float32),
            pltpu.VMEM((pts, cout), jnp.float32),
            pltpu.VMEM((_KP, cout), jnp.float32),
            pltpu.VMEM((_KP, cout), jnp.float32),
            pltpu.SemaphoreType.DMA,
            pltpu.SemaphoreType.DMA,
        ],
    )


def _conv(xp, idx, dist, w, b, c):
    bsz, n = xp.shape[0], xp.shape[1]
    cout = w.shape[0]
    wxi = w[:, :c]
    wxj = w[:, c:2 * c]
    wrel = w[:, 2 * c:2 * c + 3]
    wd = w[:, 2 * c + 3]
    wg = jnp.concatenate([wxj, wrel], axis=1).T
    wb = jnp.concatenate([wxi - wxj, -wrel], axis=1).T
    g, base = jax.vmap(
        lambda xpb: pl.pallas_call(
            _lin_body,
            out_shape=(jax.ShapeDtypeStruct((n, cout), jnp.float32),
                       jax.ShapeDtypeStruct((n, cout), jnp.float32)),
        )(xpb, wg, wb, b[None, :]))(xp)
    boff = (jnp.arange(bsz, dtype=jnp.int32) * n)[:, None, None]
    idx24 = jnp.concatenate(
        [idx, jnp.broadcast_to(idx[:, :, :1], (bsz, n, _KP - _K))],
        axis=2) + boff
    dist24 = jnp.concatenate(
        [dist, jnp.broadcast_to(dist[:, :, :1], (bsz, n, _KP - _K))], axis=2)
    out = _gmax_sc(bsz * n, cout)(
        g.reshape(bsz * n, cout),
        base.reshape(bsz * n, cout),
        idx24.reshape(bsz * n, _KP),
        dist24.reshape(bsz * n * _KP),
        wd,
    )
    return out.reshape(bsz, n, cout)


def _idis_body(dsc_ref, dsr_ref, ptT_ref, feat_ref, pos_ref, out_ref):
    n = dsc_ref.shape[0]
    n_keep = out_ref.shape[0]
    v_col = dsc_ref[:, :]
    v_row = dsr_ref[:, :]
    iota_sub = jax.lax.broadcasted_iota(jnp.int32, (n, n), 0)
    iota_lane = jax.lax.broadcasted_iota(jnp.int32, (n, n), 1)
    beats = (v_col > v_row) | ((v_col == v_row) & (iota_sub < iota_lane))
    rank = jnp.sum(beats.astype(jnp.int32), axis=0, keepdims=True)
    sel_iota = jax.lax.broadcasted_iota(jnp.int32, (n_keep, n), 0)
    sel = rank == sel_iota
    for c in range(3):
        pos_ref[:, c:c + 1] = jnp.sum(
            jnp.where(sel, ptT_ref[c:c + 1, :], 0.0), axis=1, keepdims=True)
    out_ref[:, :] = _dot(sel.astype(jnp.float32), feat_ref[:, :])


def _idis(ds, pt, feat, n_keep):
    n = pt.shape[0]
    c = feat.shape[1]
    return pl.pallas_call(
        _idis_body,
        out_shape=(jax.ShapeDtypeStruct((n_keep, 3), jnp.float32),
                   jax.ShapeDtypeStruct((n_keep, c), jnp.float32)),
    )(ds, ds.T, pt.T, feat)


def _interp_body(pq_ref, psT_ref, fs_ref, out_ref):
    nq = pq_ref.shape[0]
    m_src = psT_ref.shape[1]
    pq = pq_ref[:, :]
    psT = psT_ref[:, :]
    sqq = jnp.sum(pq * pq, axis=1, keepdims=True)
    sqs = jnp.sum(psT * psT, axis=0, keepdims=True)
    inner = _dot(pq, psT.astype(jnp.float32),
                 precision=jax.lax.Precision.DEFAULT)
    d2 = sqq + sqs - 2.0 * inner
    iota = jax.lax.broadcasted_iota(jnp.int32, (nq, m_src), 1)
    cur = d2
    acc = jnp.zeros((nq, fs_ref.shape[1]), jnp.float32)
    wsum = jnp.zeros((nq, 1), jnp.float32)
    for _ in range(3):
        mv = jnp.min(cur, axis=1, keepdims=True)
        j = jnp.min(jnp.where(cur == mv, iota, m_src), axis=1, keepdims=True)
        oh = (iota == j).astype(jnp.float32)
        nb = _dot(oh, fs_ref[:, :])
        w = 1.0 / jnp.maximum(mv, 1e-16)
        acc = acc + nb * w
        wsum = wsum + w
        cur = jnp.where(iota == j, jnp.float32(jnp.inf), cur)
    out_ref[:, :] = acc / wsum


def _interp(fs, ps, pq):
    nq = pq.shape[0]
    c = fs.shape[1]
    return pl.pallas_call(
        _interp_body,
        out_shape=jax.ShapeDtypeStruct((nq, c), jnp.float32),
    )(pq, ps.T, fs)


def _head_body(x_ref, w4_ref, b4_ref, w5_ref, b5_ref, out_ref):
    h = _leaky(_dot(x_ref[:, :], w4_ref[:, :]) + b4_ref[:, :])
    out_ref[:, :] = _dot(h, w5_ref[:, :]) + b5_ref[:, :]


def _head(x, w4, b4, w5, b5):
    n = x.shape[0]
    return pl.pallas_call(
        _head_body,
        out_shape=jax.ShapeDtypeStruct((n, w5.shape[0]), jnp.float32),
    )(x, w4.T, b4[None, :], w5.T, b5[None, :])


def kernel(x, W0, b0, W1, b1, W2, b2, W3, b3, W4, b4, W5, b5, W6, b6,
           Wc4, bc4, Wc5, bc5):
    pt = jnp.transpose(x, (0, 2, 1))
    cat = lambda *a: jnp.concatenate(a, axis=2)

    idx_a, dist_a, ds_a = jax.vmap(_knn)(pt)
    x0 = _conv(cat(pt, pt), idx_a, dist_a, W0, b0, 3)
    x1 = _conv(cat(x0, pt), idx_a, dist_a, W1, b1, 64)

    pt2, x2in = jax.vmap(lambda d, p, f: _idis(d, p, f, 512))(ds_a, pt, x1)
    idx_b, dist_b, ds_b = jax.vmap(_knn)(pt2)
    x2 = _conv(cat(x2in, pt2), idx_b, dist_b, W2, b2, 128)

    pt3, x3in = jax.vmap(lambda d, p, f: _idis(d, p, f, 256))(ds_b, pt2, x2)
    idx_c, dist_c, _ = jax.vmap(_knn)(pt3)
    x3 = _conv(cat(x3in, pt3), idx_c, dist_c, W3, b3, 256)

    i43 = jax.vmap(_interp)(x3, pt3, pt2)
    x4 = _conv(cat(i43, x2, pt2), idx_b, dist_b, W4, b4, 768)

    i54 = jax.vmap(_interp)(x4, pt2, pt)
    x5 = _conv(cat(i54, x1, pt), idx_a, dist_a, W5, b5, 384)
    x6 = _conv(cat(x5, pt), idx_a, dist_a, W6, b6, 256)

    out = jax.vmap(lambda xb: _head(xb, Wc4, bc4, Wc5, bc5))(x6)
    return jnp.transpose(out, (0, 2, 1))


# speedup vs baseline: 1.1136x; 1.0077x over previous
import functools

import jax
import jax.numpy as jnp
from jax import lax
from jax.experimental import pallas as pl
from jax.experimental.pallas import tpu as pltpu
from jax.experimental.pallas import tpu_sc as plsc

_PAR = pltpu.CompilerParams(dimension_semantics=("parallel",))
_KP = 24
_NW = 32

_K = 20
_BLK = 256


def _leaky(v):
    return jnp.where(v >= 0, v, 0.2 * v)


def _dot(a, b, precision=jax.lax.Precision.HIGHEST):
    return jax.lax.dot_general(a, b, (((1,), (0,)), ((), ())),
                               precision=precision,
                               preferred_element_type=jnp.float32)


def _knn_body(ptb_ref, ptT_ref, idx_ref, dist_ref, ds_ref):
    blk = ptb_ref.shape[0]
    n = ptT_ref.shape[1]
    ptb = ptb_ref[:, :]
    ptT = ptT_ref[:, :]
    sq_col = jnp.sum(ptb * ptb, axis=1, keepdims=True)
    sq_row = jnp.sum(ptT * ptT, axis=0, keepdims=True)
    inner = _dot(ptb, ptT, precision=jax.lax.Precision.DEFAULT)
    d2 = sq_col + sq_row - 2.0 * inner
    iota = jax.lax.broadcasted_iota(jnp.int32, (blk, n), 1)
    cur = d2
    for k in range(_K):
        m = jnp.min(cur, axis=1, keepdims=True)
        j = jnp.min(jnp.where(cur == m, iota, n), axis=1, keepdims=True)
        oh = iota == j
        dsq = jnp.zeros((blk, 1), jnp.float32)
        for c in range(3):
            pj_c = jnp.sum(jnp.where(oh, ptT[c:c + 1, :], 0.0), axis=1,
                           keepdims=True)
            rel_c = pj_c - ptb[:, c:c + 1]
            dsq = dsq + rel_c * rel_c
        dk = jnp.sqrt(jnp.maximum(dsq, 1e-12))
        idx_ref[:, k:k + 1] = j
        dist_ref[:, k:k + 1] = dk
        cur = jnp.where(oh, jnp.float32(jnp.inf), cur)
    ds_ref[:, :] = jnp.sum(dist_ref[:, :], axis=1, keepdims=True)


def _knn(pt):
    n = pt.shape[0]
    blk = min(_BLK, n)
    return pl.pallas_call(
        _knn_body,
        grid=(n // blk,),
        in_specs=[pl.BlockSpec((blk, 3), lambda i: (i, 0)),
                  pl.BlockSpec((3, n), lambda i: (0, 0))],
        out_specs=(pl.BlockSpec((blk, _K), lambda i: (i, 0)),
                   pl.BlockSpec((blk, _K), lambda i: (i, 0)),
                   pl.BlockSpec((blk, 1), lambda i: (i, 0))),
        out_shape=(jax.ShapeDtypeStruct((n, _K), jnp.int32),
                   jax.ShapeDtypeStruct((n, _K), jnp.float32),
                   jax.ShapeDtypeStruct((n, 1), jnp.float32)),
        compiler_params=_PAR,
    )(pt, pt.T)


def _lin_body(xp_ref, wg_ref, wb_ref, b_ref, g_ref, base_ref):
    xp = xp_ref[:, :]
    g_ref[:, :] = _dot(xp, wg_ref[:, :])
    base_ref[:, :] = _dot(xp, wb_ref[:, :]) + b_ref[:, :]


@functools.lru_cache(maxsize=None)
def _gmax_sc(bn, cout):
    pts = bn // _NW
    mesh = plsc.VectorSubcoreMesh(core_axis_name="c", subcore_axis_name="s")
    nchunk = cout // 16

    def body(g_hbm, base_hbm, idx_hbm, dist_hbm, wd_hbm, out_hbm,
             idx_v, dist_v, wd_v, base_v, out_v, rows_a, rows_b,
             sem_a, sem_b):
        wid = lax.axis_index("s") * 2 + lax.axis_index("c")
        p0 = wid * pts
        pltpu.sync_copy(idx_hbm.at[pl.ds(p0, pts)], idx_v.at[pl.ds(0, pts)])
        pltpu.sync_copy(idx_hbm.at[pl.ds(p0, 1)], idx_v.at[pl.ds(pts, 1)])
        pltpu.sync_copy(dist_hbm.at[pl.ds(p0 * _KP, pts * _KP)], dist_v)
        pltpu.sync_copy(wd_hbm, wd_v)
        pltpu.sync_copy(base_hbm.at[pl.ds(p0, pts)], base_v)

        def fire(p, buf, sem):
            pltpu.async_copy(g_hbm.at[idx_v.at[p]], buf, sem)

        def drain(buf, sem):
            pltpu.make_async_copy(g_hbm.at[idx_v.at[0]], buf, sem).wait()

        def compute(p, rows):
            dks = [plsc.load_gather(
                dist_v, [jnp.full((16,), p * _KP + k, jnp.int32)])
                for k in range(_KP)]
            for c in range(nchunk):
                sl = pl.ds(c * 16, 16)
                wdc = wd_v[sl]
                acc = jnp.full((16,), -jnp.inf, jnp.float32)
                for k in range(_KP):
                    acc = jnp.maximum(acc, rows[k, sl] + dks[k] * wdc)
                o = base_v[p, sl] + acc
                out_v[p, sl] = jnp.where(o >= 0, o, 0.2 * o)

        fire(0, rows_a, sem_a)

        def pair(i, carry):
            pa, pb = 2 * i, 2 * i + 1
            fire(pb, rows_b, sem_b)
            drain(rows_a, sem_a)
            compute(pa, rows_a)
            fire(pb + 1, rows_a, sem_a)
            drain(rows_b, sem_b)
            compute(pb, rows_b)
            return carry

        lax.fori_loop(0, pts // 2, pair, 0)
        drain(rows_a, sem_a)
        pltpu.sync_copy(out_v, out_hbm.at[pl.ds(p0, pts)])

    return pl.kernel(
        body,
        out_type=jax.ShapeDtypeStruct((bn, cout), jnp.float32),
        mesh=mesh,
        compiler_params=pltpu.CompilerParams(needs_layout_passes=False,
                                             use_tc_tiling_on_sc=False),
        scratch_types=[
            pltpu.VMEM((pts + 1, _KP), jnp.int32),
            pltpu.VMEM((pts * _KP,), jnp.float32),
            pltpu.VMEM((cout,), jnp.float32),
            pltpu.VMEM((pts, cout), jnp.float32),
            pltpu.VMEM((pts, cout), jnp.float32),
            pltpu.VMEM((_KP, cout), jnp.float32),
            pltpu.VMEM((_KP, cout), jnp.float32),
            pltpu.SemaphoreType.DMA,
            pltpu.SemaphoreType.DMA,
        ],
    )


def _conv(xp, idx, dist, w, b, c):
    bsz, n = xp.shape[0], xp.shape[1]
    cout = w.shape[0]
    wxi = w[:, :c]
    wxj = w[:, c:2 * c]
    wrel = w[:, 2 * c:2 * c + 3]
    wd = w[:, 2 * c + 3]
    wg = jnp.concatenate([wxj, wrel], axis=1).T
    wb = jnp.concatenate([wxi - wxj, -wrel], axis=1).T
    g, base = jax.vmap(
        lambda xpb: pl.pallas_call(
            _lin_body,
            out_shape=(jax.ShapeDtypeStruct((n, cout), jnp.float32),
                       jax.ShapeDtypeStruct((n, cout), jnp.float32)),
        )(xpb, wg, wb, b[None, :]))(xp)
    boff = (jnp.arange(bsz, dtype=jnp.int32) * n)[:, None, None]
    idx24 = jnp.concatenate(
        [idx, jnp.broadcast_to(idx[:, :, :1], (bsz, n, _KP - _K))],
        axis=2) + boff
    dist24 = jnp.concatenate(
        [dist, jnp.broadcast_to(dist[:, :, :1], (bsz, n, _KP - _K))], axis=2)
    out = _gmax_sc(bsz * n, cout)(
        g.reshape(bsz * n, cout),
        base.reshape(bsz * n, cout),
        idx24.reshape(bsz * n, _KP),
        dist24.reshape(bsz * n * _KP),
        wd,
    )
    return out.reshape(bsz, n, cout)


def _idis_body(dsc_ref, dsr_ref, ptT_ref, feat_ref, pos_ref, out_ref):
    n = dsc_ref.shape[0]
    n_keep = out_ref.shape[0]
    v_col = dsc_ref[:, :]
    v_row = dsr_ref[:, :]
    iota_sub = jax.lax.broadcasted_iota(jnp.int32, (n, n), 0)
    iota_lane = jax.lax.broadcasted_iota(jnp.int32, (n, n), 1)
    beats = (v_col > v_row) | ((v_col == v_row) & (iota_sub < iota_lane))
    rank = jnp.sum(beats.astype(jnp.int32), axis=0, keepdims=True)
    sel_iota = jax.lax.broadcasted_iota(jnp.int32, (n_keep, n), 0)
    sel = rank == sel_iota
    for c in range(3):
        pos_ref[:, c:c + 1] = jnp.sum(
            jnp.where(sel, ptT_ref[c:c + 1, :], 0.0), axis=1, keepdims=True)
    out_ref[:, :] = _dot(sel.astype(jnp.float32), feat_ref[:, :])


def _idis(ds, pt, feat, n_keep):
    n = pt.shape[0]
    c = feat.shape[1]
    return pl.pallas_call(
        _idis_body,
        out_shape=(jax.ShapeDtypeStruct((n_keep, 3), jnp.float32),
                   jax.ShapeDtypeStruct((n_keep, c), jnp.float32)),
    )(ds, ds.T, pt.T, feat)


def _interp_body(pq_ref, psT_ref, fs_ref, out_ref):
    nq = pq_ref.shape[0]
    m_src = psT_ref.shape[1]
    pq = pq_ref[:, :]
    psT = psT_ref[:, :]
    sqq = jnp.sum(pq * pq, axis=1, keepdims=True)
    sqs = jnp.sum(psT * psT, axis=0, keepdims=True)
    inner = _dot(pq, psT.astype(jnp.float32),
                 precision=jax.lax.Precision.DEFAULT)
    d2 = sqq + sqs - 2.0 * inner
    iota = jax.lax.broadcasted_iota(jnp.int32, (nq, m_src), 1)
    cur = d2
    acc = jnp.zeros((nq, fs_ref.shape[1]), jnp.float32)
    wsum = jnp.zeros((nq, 1), jnp.float32)
    for _ in range(3):
        mv = jnp.min(cur, axis=1, keepdims=True)
        j = jnp.min(jnp.where(cur == mv, iota, m_src), axis=1, keepdims=True)
        oh = (iota == j).astype(jnp.float32)
        nb = _dot(oh, fs_ref[:, :])
        w = 1.0 / jnp.maximum(mv, 1e-16)
        acc = acc + nb * w
        wsum = wsum + w
        cur = jnp.where(iota == j, jnp.float32(jnp.inf), cur)
    out_ref[:, :] = acc / wsum


def _interp(fs, ps, pq):
    nq = pq.shape[0]
    c = fs.shape[1]
    return pl.pallas_call(
        _interp_body,
        out_shape=jax.ShapeDtypeStruct((nq, c), jnp.float32),
    )(pq, ps.T, fs)


def _head_body(x_ref, w4_ref, b4_ref, w5_ref, b5_ref, out_ref):
    h = _leaky(_dot(x_ref[:, :], w4_ref[:, :]) + b4_ref[:, :])
    out_ref[:, :] = _dot(h, w5_ref[:, :]) + b5_ref[:, :]


def _head(x, w4, b4, w5, b5):
    n = x.shape[0]
    return pl.pallas_call(
        _head_body,
        out_shape=jax.ShapeDtypeStruct((n, w5.shape[0]), jnp.float32),
    )(x, w4.T, b4[None, :], w5.T, b5[None, :])


def kernel(x, W0, b0, W1, b1, W2, b2, W3, b3, W4, b4, W5, b5, W6, b6,
           Wc4, bc4, Wc5, bc5):
    pt = jnp.transpose(x, (0, 2, 1))
    cat = lambda *a: jnp.concatenate(a, axis=2)

    idx_a, dist_a, ds_a = jax.vmap(_knn)(pt)
    x0 = _conv(cat(pt, pt), idx_a, dist_a, W0, b0, 3)
    x1 = _conv(cat(x0, pt), idx_a, dist_a, W1, b1, 64)

    pt2, x2in = jax.vmap(lambda d, p, f: _idis(d, p, f, 512))(ds_a, pt, x1)
    idx_b, dist_b, ds_b = jax.vmap(_knn)(pt2)
    x2 = _conv(cat(x2in, pt2), idx_b, dist_b, W2, b2, 128)

    pt3, x3in = jax.vmap(lambda d, p, f: _idis(d, p, f, 256))(ds_b, pt2, x2)
    idx_c, dist_c, _ = jax.vmap(_knn)(pt3)
    x3 = _conv(cat(x3in, pt3), idx_c, dist_c, W3, b3, 256)

    i43 = jax.vmap(_interp)(x3, pt3, pt2)
    x4 = _conv(cat(i43, x2, pt2), idx_b, dist_b, W4, b4, 768)

    i54 = jax.vmap(_interp)(x4, pt2, pt)
    x5 = _conv(cat(i54, x1, pt), idx_a, dist_a, W5, b5, 384)
    x6 = _conv(cat(x5, pt), idx_a, dist_a, W6, b6, 256)

    out = jax.vmap(lambda xb: _head(xb, Wc4, bc4, Wc5, bc5))(x6)
    return jnp.transpose(out, (0, 2, 1))
